# baseline (speedup 1.0000x reference)
import functools

import jax
import jax.numpy as jnp
from jax import lax
from jax.experimental import pallas as pl
from jax.experimental.pallas import tpu as pltpu
from jax.experimental.pallas import tpu_sc as plsc

B, C, H, W = 32, 256, 32, 32
M, E, K = 4, 64, 2
I = E * H * W
HW = H * W
IBLK = 4096
IK = I // IBLK

NC, NS = 2, 16
NW = NC * NS
ROWS = B * M * K
RPW = ROWS // NW


def _router_body(x_ref, w_ref, loss_ref, idx_ref, gate_ref, acc_ref):
    m = pl.program_id(0)
    k = pl.program_id(1)

    @pl.when(k == 0)
    def _init():
        acc_ref[pl.ds(m, 1)] = jnp.zeros((1, B, E), jnp.float32)

    acc_ref[pl.ds(m, 1)] += jnp.dot(
        x_ref[...], w_ref[...], preferred_element_type=jnp.float32
    )[None]

    @pl.when((m == M - 1) & (k == IK - 1))
    def _route():
        logits = acc_ref[...]
        z = logits - jnp.max(logits, axis=2, keepdims=True)
        ez = jnp.exp(z)
        p = ez / jnp.sum(ez, axis=2, keepdims=True)
        iota = lax.broadcasted_iota(jnp.int32, (M, B, E), 2)
        m1 = jnp.max(p, axis=2, keepdims=True)
        i1 = jnp.min(jnp.where(p == m1, iota, E), axis=2, keepdims=True)
        mask1 = iota == i1
        p2 = jnp.where(mask1, -1.0, p)
        m2 = jnp.max(p2, axis=2, keepdims=True)
        i2 = jnp.min(jnp.where(p2 == m2, iota, E), axis=2, keepdims=True)
        denom = m1 + m2 + 1e-6
        g1 = m1 / denom
        g2 = m2 / denom
        oh1 = mask1.astype(jnp.float32)
        oh2 = (iota == i2).astype(jnp.float32)
        imp = jnp.sum(g1 * oh1 + g2 * oh2, axis=1)
        loadv = jnp.sum(oh1 + oh2, axis=1)

        def cv2(v):
            n = M * E
            s = jnp.sum(v)
            ss = jnp.sum(v * v)
            mean = s / n
            var = (ss - n * mean * mean) / (n - 1)
            return var / (mean * mean + 1e-10)

        loss_ref[...] = jnp.reshape((cv2(imp) + cv2(loadv)) * 0.01, (1, 1))
        midx = lax.broadcasted_iota(jnp.int32, (M, B, 1), 0)
        bidx = lax.broadcasted_iota(jnp.int32, (M, B, 1), 1)
        base = bidx * C + midx * E
        idx_ref[:, :, 0:1] = base + i1
        idx_ref[:, :, 1:2] = base + i2
        gate_ref[:, :, 0:1] = g1
        gate_ref[:, :, 1:2] = g2


_router = pl.pallas_call(
    _router_body,
    grid=(M, IK),
    in_specs=[
        pl.BlockSpec((B, IBLK), lambda m, k: (0, m * IK + k)),
        pl.BlockSpec((IBLK, E), lambda m, k: (m * IK + k, 0)),
    ],
    out_specs=[
        pl.BlockSpec((1, 1), lambda m, k: (0, 0)),
        pl.BlockSpec((M, B, K), lambda m, k: (0, 0, 0)),
        pl.BlockSpec((M, B, K), lambda m, k: (0, 0, 0)),
    ],
    out_shape=[
        jax.ShapeDtypeStruct((1, 1), jnp.float32),
        jax.ShapeDtypeStruct((M, B, K), jnp.int32),
        jax.ShapeDtypeStruct((M, B, K), jnp.float32),
    ],
    scratch_shapes=[pltpu.VMEM((M, B, E), jnp.float32)],
)


def _dispatch_body(xrows, arows, idx_hbm, gates_hbm, ox, oa,
                   idx_v, g_v, xr_v, ar_v, sem):
    wid = lax.axis_index("s") * NC + lax.axis_index("c")
    base = wid * RPW
    pltpu.sync_copy(idx_hbm.at[pl.ds(base, RPW)], idx_v)
    cx = pltpu.async_copy(xrows.at[idx_v], xr_v, sem)
    ca = pltpu.async_copy(arows.at[idx_v], ar_v, sem)
    pltpu.sync_copy(gates_hbm.at[pl.ds(base, RPW)], g_v)
    cx.wait()
    ca.wait()
    for r in range(RPW):
        g = g_v[r, :]

        def body(c, carry, r=r, g=g):
            sl = pl.ds(pl.multiple_of(c * 16, 16), 16)
            xr_v[r, sl] = xr_v[r, sl] * g
            ar_v[r, sl] = ar_v[r, sl] * g
            return carry

        lax.fori_loop(0, HW // 16, body, 0)
    pltpu.sync_copy(xr_v, ox.at[pl.ds(base, RPW)])
    pltpu.sync_copy(ar_v, oa.at[pl.ds(base, RPW)])


@functools.cache
def _get_dispatch():
    return functools.partial(
        pl.kernel,
        mesh=plsc.VectorSubcoreMesh(core_axis_name="c", subcore_axis_name="s"),
        out_type=[
            jax.ShapeDtypeStruct((ROWS, HW), jnp.float32),
            jax.ShapeDtypeStruct((ROWS, HW), jnp.float32),
        ],
        scratch_types=[
            pltpu.VMEM((RPW,), jnp.int32),
            pltpu.VMEM((RPW, 16), jnp.float32),
            pltpu.VMEM((RPW, HW), jnp.float32),
            pltpu.VMEM((RPW, HW), jnp.float32),
            pltpu.SemaphoreType.DMA,
        ],
    )(_dispatch_body)


def kernel(x, absolute_feature, w_gate):
    x2 = x.reshape(B, C * HW)
    w2 = w_gate.reshape(M * I, E)
    loss2, idx_mbk, gate_mbk = _router(x2, w2)
    loss = loss2[0, 0]
    idx_flat = jnp.transpose(idx_mbk, (1, 0, 2)).reshape(ROWS)
    gate_flat = jnp.transpose(gate_mbk, (1, 0, 2)).reshape(ROWS)
    gates_rep = jnp.broadcast_to(gate_flat[:, None], (ROWS, 16))
    xrows = x.reshape(B * C, HW)
    arows = absolute_feature.reshape(B * C, HW)
    ox, oa = _get_dispatch()(xrows, arows, idx_flat, gates_rep)
    wx = ox.reshape(B, M * K, H, W)
    wa = oa.reshape(B, M * K, H, W)
    return (loss, wa, wx)
